# baseline (speedup 1.0000x reference)
import jax
import jax.numpy as jnp
import numpy as np
from jax.experimental import pallas as pl
from jax.experimental.pallas import tpu as pltpu

_NINP = 32
_NH = 4
_S = 24
_B = 2
_T = 32
_DIM = 768
_NE = 16
_HID = 2048
_FFH = 3072
_CAP = 16
_THRESH = 0.2
_BAL = 0.01
_Z = 0.001
_D = 8
_F32 = jnp.float32
_FC = 6
_FW = _FFH // _FC
_NT = _B * _S * _T
_TOK = _B * _T
_NSLOT = _NE * _CAP

_P_FFB = 4
_P_EXP = _P_FFB + _FC
_P_FFA = _P_EXP + 2 * _NE
_NSTEP = _P_FFA + _FC


def _gelu(x):
    return 0.5 * x * (1.0 + jax.lax.erf(x * np.float32(0.7071067811865476)))


def _ln_lanes(x, g, b):
    mu = jnp.mean(x, axis=1, keepdims=True)
    d = x - mu
    var = jnp.mean(d * d, axis=1, keepdims=True)
    return d * jax.lax.rsqrt(var + np.float32(1e-5)) * g + b


def _attend(qkv, n, L, causal):
    scale = np.float32(1.0 / np.sqrt(_D))
    outs = []
    for h in range(_NH):
        q = jnp.swapaxes(qkv[:, h * _D:(h + 1) * _D].reshape(L, n, _D), 1, 2)
        k = jnp.swapaxes(qkv[:, _NINP + h * _D:_NINP + (h + 1) * _D].reshape(L, n, _D), 1, 2)
        v = jnp.swapaxes(qkv[:, 2 * _NINP + h * _D:2 * _NINP + (h + 1) * _D].reshape(L, n, _D), 1, 2)
        rows = []
        for i in range(L):
            s = jnp.sum(k * (q[i] * scale)[None, :, :], axis=1)
            if causal:
                mask = jax.lax.broadcasted_iota(jnp.int32, (L, n), 0) <= i
                s = jnp.where(mask, s, np.float32(-1e9))
            m = jnp.max(s, axis=0, keepdims=True)
            e = jnp.exp(s - m)
            a = e / jnp.sum(e, axis=0, keepdims=True)
            rows.append(jnp.sum(a[:, None, :] * v, axis=0)[None])
        outs.append(jnp.concatenate(rows, axis=0))
    return outs


def _mega_body(x, twt, tb, towt, tob, g1, b1, swt, sb, sowt, sob, g2, b2,
               fb1w, fb1b, fb2w, fb2b, gw, l64, lt16, e16,
               ew1, eb1, ew2, eb2, fa1w, fa1b, fa2w, fa2b, g24, g24t, g3, b3,
               y_o, aux_o,
               tx_s, sx_s, qkvt_s, qkvs_s, inp_s, xr2_s, hacc_s,
               ein_s, comb_s, eo_s, xr3_s, aglu_s,
               w1buf, w2buf, sem1, sem2):
    s = pl.program_id(0)

    @pl.when(s == 0)
    def _():
        for slot in range(2):
            pltpu.make_async_copy(ew1.at[0, :, pl.ds(slot * _HID, _HID)],
                                  w1buf.at[slot], sem1.at[slot]).start()
            pltpu.make_async_copy(ew2.at[slot], w2buf.at[slot], sem2.at[slot]).start()
        for b in range(_B):
            tx_s[:, pl.ds(b * _S, _S), :] = x[b]
            for sp in range(_S):
                sx_s[sp, pl.ds(b * _T, _T), :] = x[b, :, sp, :]
        qkvt_s[...] = (jnp.dot(tx_s[...].reshape(_NT, _NINP), twt[...],
                               preferred_element_type=_F32) + tb[...])
        qkvs_s[...] = (jnp.dot(sx_s[...].reshape(_NT, _NINP), swt[...],
                               preferred_element_type=_F32) + sb[...])

    @pl.when(s == 1)
    def _():
        heads = _attend(qkvt_s[...], 48, _T, True)
        proj = tob[...]
        for h in range(_NH):
            o2 = jnp.swapaxes(heads[h], 1, 2).reshape(_NT, _D)
            proj = proj + jnp.dot(o2, towt[pl.ds(h * _D, _D), :],
                                  preferred_element_type=_F32)
        tm = _ln_lanes(proj + tx_s[...].reshape(_NT, _NINP), g1[...], b1[...])
        tm3 = tm.reshape(_T, 48, _NINP)
        for b in range(_B):
            for sp in range(_S):
                inp_s[pl.ds(b * _T, _T), pl.ds(sp * _NINP, _NINP)] = tm3[:, b * _S + sp, :]

    @pl.when(s == 2)
    def _():
        heads = _attend(qkvs_s[...], 64, _S, False)
        proj = sob[...]
        for h in range(_NH):
            o2 = jnp.swapaxes(heads[h], 1, 2).reshape(_NT, _D)
            proj = proj + jnp.dot(o2, sowt[pl.ds(h * _D, _D), :],
                                  preferred_element_type=_F32)
        sm = _ln_lanes(proj + sx_s[...].reshape(_NT, _NINP), g2[...], b2[...])
        sm3 = sm.reshape(_S, _TOK, _NINP)
        for sp in range(_S):
            inp_s[:, pl.ds(sp * _NINP, _NINP)] += sm3[sp]

    @pl.when(s == 3)
    def _():
        hacc_s[...] = jnp.zeros_like(hacc_s)

    @pl.when((s >= _P_FFB) & (s < _P_FFB + _FC))
    def _():
        h = _gelu(jnp.dot(inp_s[...], fb1w[...], preferred_element_type=_F32)
                  + fb1b[...])
        hacc_s[...] += jnp.dot(h, fb2w[...], preferred_element_type=_F32)

    @pl.when(s == _P_FFB + _FC - 1)
    def _():
        inp = inp_s[...]
        xr2 = inp + hacc_s[...] + fb2b[...]
        xr2_s[...] = xr2

        logits = jnp.dot(xr2, gw[...], preferred_element_type=_F32)
        mx = jnp.max(logits, axis=1, keepdims=True)
        ex = jnp.exp(logits - mx)
        se = jnp.sum(ex, axis=1, keepdims=True)
        probs = ex / se
        lse = mx + jnp.log(se)
        zl = jnp.mean(lse * lse) * np.float32(_Z)

        v1 = jnp.max(probs, axis=1, keepdims=True)
        m1r = (probs == v1).astype(_F32)
        c1 = jnp.dot(m1r, lt16[...], preferred_element_type=_F32)
        m1 = m1r * (c1 == 1.0).astype(_F32)
        probs2 = probs * (1.0 - m1)
        v2 = jnp.max(probs2, axis=1, keepdims=True)
        m2r = (probs2 == v2).astype(_F32)
        c2 = jnp.dot(m2r, lt16[...], preferred_element_type=_F32)
        m2 = m2r * (c2 == 1.0).astype(_F32) * (v2 > np.float32(_THRESH)).astype(_F32)

        density = jnp.mean(probs, axis=0, keepdims=True)
        d1m = jnp.mean(m1, axis=0, keepdims=True)
        bal = jnp.mean(density * d1m) * np.float32(_NE * _NE * _BAL)
        aux_o[...] = jnp.broadcast_to(bal + zl, (1, 1))

        pos1 = jnp.dot(l64[...], m1, preferred_element_type=_F32) - 1.0
        m1k = m1 * (pos1 < np.float32(_CAP)).astype(_F32)
        cnt1 = jnp.sum(m1, axis=0, keepdims=True)
        pos2 = jnp.dot(l64[...], m2, preferred_element_type=_F32) - 1.0 + cnt1
        m2k = m2 * (pos2 < np.float32(_CAP)).astype(_F32)

        e16v = e16[...]
        ci = (jax.lax.broadcasted_iota(jnp.int32, (_TOK, _NSLOT), 1) % _CAP
              ).astype(_F32)
        oh1 = (jnp.dot(pos1, e16v, preferred_element_type=_F32) == ci).astype(_F32)
        oh2 = (jnp.dot(pos2, e16v, preferred_element_type=_F32) == ci).astype(_F32)
        d1e = jnp.dot(m1k, e16v, preferred_element_type=_F32) * oh1
        d2e = jnp.dot(m2k, e16v, preferred_element_type=_F32) * oh2
        comb_s[...] = v1 * d1e + v2 * d2e
        disp = d1e + d2e
        ein_s[...] = jax.lax.dot_general(disp, xr2, (((0,), (0,)), ((), ())),
                                         preferred_element_type=_F32)

    @pl.when((s >= _P_EXP) & (s < _P_EXP + 2 * _NE))
    def _():
        sp = s - _P_EXP
        e = jax.lax.div(sp, 2)
        hh = jax.lax.rem(sp, 2)
        slot = jax.lax.rem(sp, 2)
        row = pl.multiple_of(e * _CAP, _CAP)
        ein_e = ein_s[pl.ds(row, _CAP), :]

        pltpu.make_async_copy(ew1.at[0, :, pl.ds(0, _HID)],
                              w1buf.at[slot], sem1.at[slot]).wait()
        part = jnp.dot(ein_e, w1buf[slot], preferred_element_type=_F32)

        @pl.when(hh == 0)
        def _():
            aglu_s[...] = part + eb1[pl.ds(e, 1), pl.ds(0, _HID)]

        @pl.when(hh == 1)
        def _():
            g = part + eb1[pl.ds(e, 1), pl.ds(_HID, _HID)]
            act = aglu_s[...] * _gelu(g)
            pltpu.make_async_copy(ew2.at[0], w2buf.at[jax.lax.rem(e, 2)],
                                  sem2.at[jax.lax.rem(e, 2)]).wait()
            eo = (jnp.dot(act, w2buf[jax.lax.rem(e, 2)],
                          preferred_element_type=_F32) + eb2[pl.ds(e, 1)])
            eo_s[pl.ds(row, _CAP), :] = eo

            @pl.when(e + 2 < _NE)
            def _():
                pltpu.make_async_copy(ew2.at[e + 2], w2buf.at[jax.lax.rem(e, 2)],
                                      sem2.at[jax.lax.rem(e, 2)]).start()

        @pl.when(sp + 2 < 2 * _NE)
        def _():
            e2 = jax.lax.div(sp + 2, 2)
            h2 = jax.lax.rem(sp + 2, 2)
            pltpu.make_async_copy(ew1.at[e2, :, pl.ds(h2 * _HID, _HID)],
                                  w1buf.at[slot], sem1.at[slot]).start()

    @pl.when(s == _P_FFA)
    def _():
        xr3_s[...] = xr2_s[...] + jnp.dot(comb_s[...], eo_s[...],
                                          preferred_element_type=_F32)
        hacc_s[...] = jnp.zeros_like(hacc_s)

    @pl.when(s >= _P_FFA)
    def _():
        h = _gelu(jnp.dot(xr3_s[...], fa1w[...], preferred_element_type=_F32)
                  + fa1b[...])
        hacc_s[...] += jnp.dot(h, fa2w[...], preferred_element_type=_F32)

    @pl.when(s == _NSTEP - 1)
    def _():
        z = xr3_s[...] + hacc_s[...] + fa2b[...] + inp_s[...]
        inv = np.float32(1.0 / _NINP)
        mu = jnp.dot(jnp.dot(z, g24[...], preferred_element_type=_F32) * inv,
                     g24t[...], preferred_element_type=_F32)
        d = z - mu
        var = jnp.dot(jnp.dot(d * d, g24[...], preferred_element_type=_F32) * inv,
                      g24t[...], preferred_element_type=_F32)
        y_o[...] = d * jax.lax.rsqrt(var + np.float32(1e-5)) * g3[...] + b3[...]


def kernel(x, t_in_w, t_in_b, t_out_w, t_out_b, s_in_w, s_in_b, s_out_w, s_out_b,
           ln1_g, ln1_b, ln2_g, ln2_b, ln3_g, ln3_b,
           ffb_w1, ffb_b1, ffb_w2, ffb_b2,
           gate_w, ew1, eb1, ew2, eb2,
           ffa_w1, ffa_b1, ffa_w2, ffa_b2):
    f32 = _F32

    l64 = jnp.tril(jnp.ones((_TOK, _TOK), f32))
    lt16 = jnp.triu(jnp.ones((_NE, _NE), f32))
    e16 = (jnp.arange(_NSLOT, dtype=jnp.int32)[None, :] // _CAP ==
           jnp.arange(_NE, dtype=jnp.int32)[:, None]).astype(f32)
    g24 = (jnp.arange(_DIM, dtype=jnp.int32)[:, None] // _NINP ==
           jnp.arange(_S, dtype=jnp.int32)[None, :]).astype(f32)
    g3 = jnp.tile(ln3_g, _S).reshape(1, _DIM)
    b3 = jnp.tile(ln3_b, _S).reshape(1, _DIM)

    cst = lambda *idx: (lambda s, _i=idx: _i)
    ffb_i = lambda s: (0, jnp.clip(s - _P_FFB, 0, _FC - 1))
    ffb_i2 = lambda s: (jnp.clip(s - _P_FFB, 0, _FC - 1), 0)
    ffa_i = lambda s: (0, jnp.clip(s - _P_FFA, 0, _FC - 1))
    ffa_i2 = lambda s: (jnp.clip(s - _P_FFA, 0, _FC - 1), 0)

    y, aux = pl.pallas_call(
        _mega_body,
        grid=(_NSTEP,),
        in_specs=[
            pl.BlockSpec((_B, _T, _S, _NINP), cst(0, 0, 0, 0)),
            pl.BlockSpec((_NINP, 3 * _NINP), cst(0, 0)),
            pl.BlockSpec((1, 3 * _NINP), cst(0, 0)),
            pl.BlockSpec((_NINP, _NINP), cst(0, 0)),
            pl.BlockSpec((1, _NINP), cst(0, 0)),
            pl.BlockSpec((1, _NINP), cst(0, 0)),
            pl.BlockSpec((1, _NINP), cst(0, 0)),
            pl.BlockSpec((_NINP, 3 * _NINP), cst(0, 0)),
            pl.BlockSpec((1, 3 * _NINP), cst(0, 0)),
            pl.BlockSpec((_NINP, _NINP), cst(0, 0)),
            pl.BlockSpec((1, _NINP), cst(0, 0)),
            pl.BlockSpec((1, _NINP), cst(0, 0)),
            pl.BlockSpec((1, _NINP), cst(0, 0)),
            pl.BlockSpec((_DIM, _FW), ffb_i),
            pl.BlockSpec((1, _FW), ffb_i),
            pl.BlockSpec((_FW, _DIM), ffb_i2),
            pl.BlockSpec((1, _DIM), cst(0, 0)),
            pl.BlockSpec((_DIM, _NE), cst(0, 0)),
            pl.BlockSpec((_TOK, _TOK), cst(0, 0)),
            pl.BlockSpec((_NE, _NE), cst(0, 0)),
            pl.BlockSpec((_NE, _NSLOT), cst(0, 0)),
            pl.BlockSpec(memory_space=pl.ANY),
            pl.BlockSpec((_NE, 2 * _HID), cst(0, 0)),
            pl.BlockSpec(memory_space=pl.ANY),
            pl.BlockSpec((_NE, _DIM), cst(0, 0)),
            pl.BlockSpec((_DIM, _FW), ffa_i),
            pl.BlockSpec((1, _FW), ffa_i),
            pl.BlockSpec((_FW, _DIM), ffa_i2),
            pl.BlockSpec((1, _DIM), cst(0, 0)),
            pl.BlockSpec((_DIM, _S), cst(0, 0)),
            pl.BlockSpec((_S, _DIM), cst(0, 0)),
            pl.BlockSpec((1, _DIM), cst(0, 0)),
            pl.BlockSpec((1, _DIM), cst(0, 0)),
        ],
        out_specs=[pl.BlockSpec((_TOK, _DIM), cst(0, 0)),
                   pl.BlockSpec((1, 1), cst(0, 0))],
        out_shape=[jax.ShapeDtypeStruct((_TOK, _DIM), f32),
                   jax.ShapeDtypeStruct((1, 1), f32)],
        scratch_shapes=[
            pltpu.VMEM((_T, 48, _NINP), f32),
            pltpu.VMEM((_S, _TOK, _NINP), f32),
            pltpu.VMEM((_NT, 3 * _NINP), f32),
            pltpu.VMEM((_NT, 3 * _NINP), f32),
            pltpu.VMEM((_TOK, _DIM), f32),
            pltpu.VMEM((_TOK, _DIM), f32),
            pltpu.VMEM((_TOK, _DIM), f32),
            pltpu.VMEM((_NSLOT, _DIM), f32),
            pltpu.VMEM((_TOK, _NSLOT), f32),
            pltpu.VMEM((_NSLOT, _DIM), f32),
            pltpu.VMEM((_TOK, _DIM), f32),
            pltpu.VMEM((_CAP, _HID), f32),
            pltpu.VMEM((2, _DIM, _HID), f32),
            pltpu.VMEM((2, _HID, _DIM), f32),
            pltpu.SemaphoreType.DMA((2,)),
            pltpu.SemaphoreType.DMA((2,)),
        ],
    )(x, t_in_w.T, t_in_b.reshape(1, -1), t_out_w.T, t_out_b.reshape(1, -1),
      ln1_g.reshape(1, -1), ln1_b.reshape(1, -1),
      s_in_w.T, s_in_b.reshape(1, -1), s_out_w.T, s_out_b.reshape(1, -1),
      ln2_g.reshape(1, -1), ln2_b.reshape(1, -1),
      ffb_w1, ffb_b1.reshape(1, -1), ffb_w2, ffb_b2.reshape(1, -1),
      gate_w, l64, lt16, e16,
      ew1, eb1, ew2, eb2,
      ffa_w1, ffa_b1.reshape(1, -1), ffa_w2, ffa_b2.reshape(1, -1),
      g24, g24.T, g3, b3)

    return y.reshape(_B, _T, _S, _NINP), aux[0, 0]


# speedup vs baseline: 1.0336x; 1.0336x over previous
import jax
import jax.numpy as jnp
import numpy as np
from jax.experimental import pallas as pl
from jax.experimental.pallas import tpu as pltpu

_NINP = 32
_NH = 4
_S = 24
_B = 2
_T = 32
_DIM = 768
_NE = 16
_HID = 2048
_FFH = 3072
_CAP = 16
_THRESH = 0.2
_BAL = 0.01
_Z = 0.001
_D = 8
_F32 = jnp.float32
_FC = 6
_FW = _FFH // _FC
_NT = _B * _S * _T
_TOK = _B * _T
_NSLOT = _NE * _CAP

_P_FFB = 4
_P_EXP = _P_FFB + _FC
_P_FFA = _P_EXP + 2 * _NE
_NSTEP = _P_FFA + _FC
_NS1 = 4
_NS2 = 2


def _gelu(x):
    return 0.5 * x * (1.0 + jax.lax.erf(x * np.float32(0.7071067811865476)))


def _ln_lanes(x, g, b):
    mu = jnp.mean(x, axis=1, keepdims=True)
    d = x - mu
    var = jnp.mean(d * d, axis=1, keepdims=True)
    return d * jax.lax.rsqrt(var + np.float32(1e-5)) * g + b


def _attend(qkv, n, L, causal):
    scale = np.float32(1.0 / np.sqrt(_D))
    outs = []
    for h in range(_NH):
        q = jnp.swapaxes(qkv[:, h * _D:(h + 1) * _D].reshape(L, n, _D), 1, 2)
        k = jnp.swapaxes(qkv[:, _NINP + h * _D:_NINP + (h + 1) * _D].reshape(L, n, _D), 1, 2)
        v = jnp.swapaxes(qkv[:, 2 * _NINP + h * _D:2 * _NINP + (h + 1) * _D].reshape(L, n, _D), 1, 2)
        rows = []
        for i in range(L):
            s = jnp.sum(k * (q[i] * scale)[None, :, :], axis=1)
            if causal:
                mask = jax.lax.broadcasted_iota(jnp.int32, (L, n), 0) <= i
                s = jnp.where(mask, s, np.float32(-1e9))
            m = jnp.max(s, axis=0, keepdims=True)
            e = jnp.exp(s - m)
            a = e / jnp.sum(e, axis=0, keepdims=True)
            rows.append(jnp.sum(a[:, None, :] * v, axis=0)[None])
        outs.append(jnp.concatenate(rows, axis=0))
    return outs


def _mega_body(x, twt, tb, towt, tob, g1, b1, swt, sb, sowt, sob, g2, b2,
               fb1w, fb1b, fb2w, fb2b, gw, l64, lt16, e16,
               ew1, eb1, ew2, eb2, fa1w, fa1b, fa2w, fa2b, g24, g24t, g3, b3,
               y_o, aux_o,
               tx_s, sx_s, qkvt_s, qkvs_s, inp_s, xr2_s, hacc_s,
               ein_s, comb_s, eo_s, xr3_s, aglu_s,
               w1buf, w2buf, sem1, sem2):
    s = pl.program_id(0)

    @pl.when(s == 0)
    def _():
        for slot in range(_NS1):
            pltpu.make_async_copy(
                ew1.at[slot // 2, :, pl.ds((slot % 2) * _HID, _HID)],
                w1buf.at[slot], sem1.at[slot]).start()
        for slot in range(_NS2):
            pltpu.make_async_copy(ew2.at[slot], w2buf.at[slot], sem2.at[slot]).start()
        for b in range(_B):
            tx_s[:, pl.ds(b * _S, _S), :] = x[b]
            for sp in range(_S):
                sx_s[sp, pl.ds(b * _T, _T), :] = x[b, :, sp, :]
        qkvt_s[...] = (jnp.dot(tx_s[...].reshape(_NT, _NINP), twt[...],
                               preferred_element_type=_F32) + tb[...])
        qkvs_s[...] = (jnp.dot(sx_s[...].reshape(_NT, _NINP), swt[...],
                               preferred_element_type=_F32) + sb[...])

    @pl.when(s == 1)
    def _():
        heads = _attend(qkvt_s[...], 48, _T, True)
        proj = tob[...]
        for h in range(_NH):
            o2 = jnp.swapaxes(heads[h], 1, 2).reshape(_NT, _D)
            proj = proj + jnp.dot(o2, towt[pl.ds(h * _D, _D), :],
                                  preferred_element_type=_F32)
        tm = _ln_lanes(proj + tx_s[...].reshape(_NT, _NINP), g1[...], b1[...])
        tm3 = tm.reshape(_T, 48, _NINP)
        for b in range(_B):
            for sp in range(_S):
                inp_s[pl.ds(b * _T, _T), pl.ds(sp * _NINP, _NINP)] = tm3[:, b * _S + sp, :]

    @pl.when(s == 2)
    def _():
        heads = _attend(qkvs_s[...], 64, _S, False)
        proj = sob[...]
        for h in range(_NH):
            o2 = jnp.swapaxes(heads[h], 1, 2).reshape(_NT, _D)
            proj = proj + jnp.dot(o2, sowt[pl.ds(h * _D, _D), :],
                                  preferred_element_type=_F32)
        sm = _ln_lanes(proj + sx_s[...].reshape(_NT, _NINP), g2[...], b2[...])
        sm3 = sm.reshape(_S, _TOK, _NINP)
        for sp in range(_S):
            inp_s[:, pl.ds(sp * _NINP, _NINP)] += sm3[sp]

    @pl.when(s == 3)
    def _():
        hacc_s[...] = jnp.zeros_like(hacc_s)

    @pl.when((s >= _P_FFB) & (s < _P_FFB + _FC))
    def _():
        h = _gelu(jnp.dot(inp_s[...], fb1w[...], preferred_element_type=_F32)
                  + fb1b[...])
        hacc_s[...] += jnp.dot(h, fb2w[...], preferred_element_type=_F32)

    @pl.when(s == _P_FFB + _FC - 1)
    def _():
        inp = inp_s[...]
        xr2 = inp + hacc_s[...] + fb2b[...]
        xr2_s[...] = xr2

        logits = jnp.dot(xr2, gw[...], preferred_element_type=_F32)
        mx = jnp.max(logits, axis=1, keepdims=True)
        ex = jnp.exp(logits - mx)
        se = jnp.sum(ex, axis=1, keepdims=True)
        probs = ex / se
        lse = mx + jnp.log(se)
        zl = jnp.mean(lse * lse) * np.float32(_Z)

        v1 = jnp.max(probs, axis=1, keepdims=True)
        m1r = (probs == v1).astype(_F32)
        c1 = jnp.dot(m1r, lt16[...], preferred_element_type=_F32)
        m1 = m1r * (c1 == 1.0).astype(_F32)
        probs2 = probs * (1.0 - m1)
        v2 = jnp.max(probs2, axis=1, keepdims=True)
        m2r = (probs2 == v2).astype(_F32)
        c2 = jnp.dot(m2r, lt16[...], preferred_element_type=_F32)
        m2 = m2r * (c2 == 1.0).astype(_F32) * (v2 > np.float32(_THRESH)).astype(_F32)

        density = jnp.mean(probs, axis=0, keepdims=True)
        d1m = jnp.mean(m1, axis=0, keepdims=True)
        bal = jnp.mean(density * d1m) * np.float32(_NE * _NE * _BAL)
        aux_o[...] = jnp.broadcast_to(bal + zl, (1, 1))

        pos1 = jnp.dot(l64[...], m1, preferred_element_type=_F32) - 1.0
        m1k = m1 * (pos1 < np.float32(_CAP)).astype(_F32)
        cnt1 = jnp.sum(m1, axis=0, keepdims=True)
        pos2 = jnp.dot(l64[...], m2, preferred_element_type=_F32) - 1.0 + cnt1
        m2k = m2 * (pos2 < np.float32(_CAP)).astype(_F32)

        e16v = e16[...]
        ci = (jax.lax.broadcasted_iota(jnp.int32, (_TOK, _NSLOT), 1) % _CAP
              ).astype(_F32)
        oh1 = (jnp.dot(pos1, e16v, preferred_element_type=_F32) == ci).astype(_F32)
        oh2 = (jnp.dot(pos2, e16v, preferred_element_type=_F32) == ci).astype(_F32)
        d1e = jnp.dot(m1k, e16v, preferred_element_type=_F32) * oh1
        d2e = jnp.dot(m2k, e16v, preferred_element_type=_F32) * oh2
        comb_s[...] = v1 * d1e + v2 * d2e
        disp = d1e + d2e
        ein_s[...] = jax.lax.dot_general(disp, xr2, (((0,), (0,)), ((), ())),
                                         preferred_element_type=_F32)

    @pl.when((s >= _P_EXP) & (s < _P_EXP + 2 * _NE))
    def _():
        sp = s - _P_EXP
        e = jax.lax.div(sp, 2)
        hh = jax.lax.rem(sp, 2)
        slot = jax.lax.rem(sp, _NS1)
        slot2 = jax.lax.rem(e, _NS2)
        row = pl.multiple_of(e * _CAP, _CAP)
        ein_e = ein_s[pl.ds(row, _CAP), :]

        pltpu.make_async_copy(ew1.at[0, :, pl.ds(0, _HID)],
                              w1buf.at[slot], sem1.at[slot]).wait()
        part = jnp.dot(ein_e, w1buf[slot], preferred_element_type=_F32)

        @pl.when(hh == 0)
        def _():
            aglu_s[...] = part + eb1[pl.ds(e, 1), pl.ds(0, _HID)]

        @pl.when(hh == 1)
        def _():
            g = part + eb1[pl.ds(e, 1), pl.ds(_HID, _HID)]
            act = aglu_s[...] * _gelu(g)
            pltpu.make_async_copy(ew2.at[0], w2buf.at[slot2],
                                  sem2.at[slot2]).wait()
            eo = (jnp.dot(act, w2buf[slot2],
                          preferred_element_type=_F32) + eb2[pl.ds(e, 1)])
            eo_s[pl.ds(row, _CAP), :] = eo

            @pl.when(e + _NS2 < _NE)
            def _():
                pltpu.make_async_copy(ew2.at[e + _NS2], w2buf.at[slot2],
                                      sem2.at[slot2]).start()

        @pl.when(sp + _NS1 < 2 * _NE)
        def _():
            e2 = jax.lax.div(sp + _NS1, 2)
            h2 = jax.lax.rem(sp + _NS1, 2)
            pltpu.make_async_copy(ew1.at[e2, :, pl.ds(h2 * _HID, _HID)],
                                  w1buf.at[slot], sem1.at[slot]).start()

    @pl.when(s == _P_FFA)
    def _():
        xr3_s[...] = xr2_s[...] + jnp.dot(comb_s[...], eo_s[...],
                                          preferred_element_type=_F32)
        hacc_s[...] = jnp.zeros_like(hacc_s)

    @pl.when(s >= _P_FFA)
    def _():
        h = _gelu(jnp.dot(xr3_s[...], fa1w[...], preferred_element_type=_F32)
                  + fa1b[...])
        hacc_s[...] += jnp.dot(h, fa2w[...], preferred_element_type=_F32)

    @pl.when(s == _NSTEP - 1)
    def _():
        z = xr3_s[...] + hacc_s[...] + fa2b[...] + inp_s[...]
        inv = np.float32(1.0 / _NINP)
        mu = jnp.dot(jnp.dot(z, g24[...], preferred_element_type=_F32) * inv,
                     g24t[...], preferred_element_type=_F32)
        d = z - mu
        var = jnp.dot(jnp.dot(d * d, g24[...], preferred_element_type=_F32) * inv,
                      g24t[...], preferred_element_type=_F32)
        y_o[...] = d * jax.lax.rsqrt(var + np.float32(1e-5)) * g3[...] + b3[...]


def kernel(x, t_in_w, t_in_b, t_out_w, t_out_b, s_in_w, s_in_b, s_out_w, s_out_b,
           ln1_g, ln1_b, ln2_g, ln2_b, ln3_g, ln3_b,
           ffb_w1, ffb_b1, ffb_w2, ffb_b2,
           gate_w, ew1, eb1, ew2, eb2,
           ffa_w1, ffa_b1, ffa_w2, ffa_b2):
    f32 = _F32

    l64 = jnp.tril(jnp.ones((_TOK, _TOK), f32))
    lt16 = jnp.triu(jnp.ones((_NE, _NE), f32))
    e16 = (jnp.arange(_NSLOT, dtype=jnp.int32)[None, :] // _CAP ==
           jnp.arange(_NE, dtype=jnp.int32)[:, None]).astype(f32)
    g24 = (jnp.arange(_DIM, dtype=jnp.int32)[:, None] // _NINP ==
           jnp.arange(_S, dtype=jnp.int32)[None, :]).astype(f32)
    g3 = jnp.tile(ln3_g, _S).reshape(1, _DIM)
    b3 = jnp.tile(ln3_b, _S).reshape(1, _DIM)

    cst = lambda *idx: (lambda s, _i=idx: _i)
    ffb_i = lambda s: (0, jnp.clip(s - _P_FFB, 0, _FC - 1))
    ffb_i2 = lambda s: (jnp.clip(s - _P_FFB, 0, _FC - 1), 0)
    ffa_i = lambda s: (0, jnp.clip(s - _P_FFA, 0, _FC - 1))
    ffa_i2 = lambda s: (jnp.clip(s - _P_FFA, 0, _FC - 1), 0)

    y, aux = pl.pallas_call(
        _mega_body,
        grid=(_NSTEP,),
        compiler_params=pltpu.CompilerParams(
            vmem_limit_bytes=100 * 1024 * 1024),
        in_specs=[
            pl.BlockSpec((_B, _T, _S, _NINP), cst(0, 0, 0, 0)),
            pl.BlockSpec((_NINP, 3 * _NINP), cst(0, 0)),
            pl.BlockSpec((1, 3 * _NINP), cst(0, 0)),
            pl.BlockSpec((_NINP, _NINP), cst(0, 0)),
            pl.BlockSpec((1, _NINP), cst(0, 0)),
            pl.BlockSpec((1, _NINP), cst(0, 0)),
            pl.BlockSpec((1, _NINP), cst(0, 0)),
            pl.BlockSpec((_NINP, 3 * _NINP), cst(0, 0)),
            pl.BlockSpec((1, 3 * _NINP), cst(0, 0)),
            pl.BlockSpec((_NINP, _NINP), cst(0, 0)),
            pl.BlockSpec((1, _NINP), cst(0, 0)),
            pl.BlockSpec((1, _NINP), cst(0, 0)),
            pl.BlockSpec((1, _NINP), cst(0, 0)),
            pl.BlockSpec((_DIM, _FW), ffb_i),
            pl.BlockSpec((1, _FW), ffb_i),
            pl.BlockSpec((_FW, _DIM), ffb_i2),
            pl.BlockSpec((1, _DIM), cst(0, 0)),
            pl.BlockSpec((_DIM, _NE), cst(0, 0)),
            pl.BlockSpec((_TOK, _TOK), cst(0, 0)),
            pl.BlockSpec((_NE, _NE), cst(0, 0)),
            pl.BlockSpec((_NE, _NSLOT), cst(0, 0)),
            pl.BlockSpec(memory_space=pl.ANY),
            pl.BlockSpec((_NE, 2 * _HID), cst(0, 0)),
            pl.BlockSpec(memory_space=pl.ANY),
            pl.BlockSpec((_NE, _DIM), cst(0, 0)),
            pl.BlockSpec((_DIM, _FW), ffa_i),
            pl.BlockSpec((1, _FW), ffa_i),
            pl.BlockSpec((_FW, _DIM), ffa_i2),
            pl.BlockSpec((1, _DIM), cst(0, 0)),
            pl.BlockSpec((_DIM, _S), cst(0, 0)),
            pl.BlockSpec((_S, _DIM), cst(0, 0)),
            pl.BlockSpec((1, _DIM), cst(0, 0)),
            pl.BlockSpec((1, _DIM), cst(0, 0)),
        ],
        out_specs=[pl.BlockSpec((_TOK, _DIM), cst(0, 0)),
                   pl.BlockSpec((1, 1), cst(0, 0))],
        out_shape=[jax.ShapeDtypeStruct((_TOK, _DIM), f32),
                   jax.ShapeDtypeStruct((1, 1), f32)],
        scratch_shapes=[
            pltpu.VMEM((_T, 48, _NINP), f32),
            pltpu.VMEM((_S, _TOK, _NINP), f32),
            pltpu.VMEM((_NT, 3 * _NINP), f32),
            pltpu.VMEM((_NT, 3 * _NINP), f32),
            pltpu.VMEM((_TOK, _DIM), f32),
            pltpu.VMEM((_TOK, _DIM), f32),
            pltpu.VMEM((_TOK, _DIM), f32),
            pltpu.VMEM((_NSLOT, _DIM), f32),
            pltpu.VMEM((_TOK, _NSLOT), f32),
            pltpu.VMEM((_NSLOT, _DIM), f32),
            pltpu.VMEM((_TOK, _DIM), f32),
            pltpu.VMEM((_CAP, _HID), f32),
            pltpu.VMEM((_NS1, _DIM, _HID), f32),
            pltpu.VMEM((_NS2, _HID, _DIM), f32),
            pltpu.SemaphoreType.DMA((_NS1,)),
            pltpu.SemaphoreType.DMA((_NS2,)),
        ],
    )(x, t_in_w.T, t_in_b.reshape(1, -1), t_out_w.T, t_out_b.reshape(1, -1),
      ln1_g.reshape(1, -1), ln1_b.reshape(1, -1),
      s_in_w.T, s_in_b.reshape(1, -1), s_out_w.T, s_out_b.reshape(1, -1),
      ln2_g.reshape(1, -1), ln2_b.reshape(1, -1),
      ffb_w1, ffb_b1.reshape(1, -1), ffb_w2, ffb_b2.reshape(1, -1),
      gate_w, l64, lt16, e16,
      ew1, eb1, ew2, eb2,
      ffa_w1, ffa_b1.reshape(1, -1), ffa_w2, ffa_b2.reshape(1, -1),
      g24, g24.T, g3, b3)

    return y.reshape(_B, _T, _S, _NINP), aux[0, 0]


# speedup vs baseline: 1.0350x; 1.0013x over previous
import jax
import jax.numpy as jnp
import numpy as np
from jax.experimental import pallas as pl
from jax.experimental.pallas import tpu as pltpu

_NINP = 32
_NH = 4
_S = 24
_B = 2
_T = 32
_DIM = 768
_NE = 16
_HID = 2048
_FFH = 3072
_CAP = 16
_THRESH = 0.2
_BAL = 0.01
_Z = 0.001
_D = 8
_F32 = jnp.float32
_FC = 6
_FW = _FFH // _FC
_NT = _B * _S * _T
_TOK = _B * _T
_NSLOT = _NE * _CAP

_P_FFB = 4
_P_EXP = _P_FFB + _FC
_P_FFA = _P_EXP + _NE
_NSTEP = _P_FFA + _FC
_NS1 = 2
_NS2 = 2


def _gelu(x):
    return 0.5 * x * (1.0 + jax.lax.erf(x * np.float32(0.7071067811865476)))


def _ln_lanes(x, g, b):
    mu = jnp.mean(x, axis=1, keepdims=True)
    d = x - mu
    var = jnp.mean(d * d, axis=1, keepdims=True)
    return d * jax.lax.rsqrt(var + np.float32(1e-5)) * g + b


def _attend(qkv, n, L, causal):
    scale = np.float32(1.0 / np.sqrt(_D))
    outs = []
    for h in range(_NH):
        q = jnp.swapaxes(qkv[:, h * _D:(h + 1) * _D].reshape(L, n, _D), 1, 2)
        k = jnp.swapaxes(qkv[:, _NINP + h * _D:_NINP + (h + 1) * _D].reshape(L, n, _D), 1, 2)
        v = jnp.swapaxes(qkv[:, 2 * _NINP + h * _D:2 * _NINP + (h + 1) * _D].reshape(L, n, _D), 1, 2)
        rows = []
        for i in range(L):
            s = jnp.sum(k * (q[i] * scale)[None, :, :], axis=1)
            if causal:
                mask = jax.lax.broadcasted_iota(jnp.int32, (L, n), 0) <= i
                s = jnp.where(mask, s, np.float32(-1e9))
            m = jnp.max(s, axis=0, keepdims=True)
            e = jnp.exp(s - m)
            a = e / jnp.sum(e, axis=0, keepdims=True)
            rows.append(jnp.sum(a[:, None, :] * v, axis=0)[None])
        outs.append(jnp.concatenate(rows, axis=0))
    return outs


def _mega_body(x, twt, tb, towt, tob, g1, b1, swt, sb, sowt, sob, g2, b2,
               fb1w, fb1b, fb2w, fb2b, gw, l64, lt16, e16,
               ew1, eb1, ew2, eb2, fa1w, fa1b, fa2w, fa2b, g24, g24t, g3, b3,
               y_o, aux_o,
               tx_s, sx_s, qkvt_s, qkvs_s, inp_s, xr2_s, hacc_s,
               ein_s, comb_s, eo_s, xr3_s, aglu_s,
               w1buf, w2buf, sem1, sem2):
    s = pl.program_id(0)

    @pl.when(s == 0)
    def _():
        for slot in range(_NS1):
            pltpu.make_async_copy(ew1.at[slot], w1buf.at[slot], sem1.at[slot]).start()
        for slot in range(_NS2):
            pltpu.make_async_copy(ew2.at[slot], w2buf.at[slot], sem2.at[slot]).start()
        for b in range(_B):
            tx_s[:, pl.ds(b * _S, _S), :] = x[b]
            for sp in range(_S):
                sx_s[sp, pl.ds(b * _T, _T), :] = x[b, :, sp, :]
        qkvt_s[...] = (jnp.dot(tx_s[...].reshape(_NT, _NINP), twt[...],
                               preferred_element_type=_F32) + tb[...])
        qkvs_s[...] = (jnp.dot(sx_s[...].reshape(_NT, _NINP), swt[...],
                               preferred_element_type=_F32) + sb[...])

    @pl.when(s == 1)
    def _():
        heads = _attend(qkvt_s[...], 48, _T, True)
        proj = tob[...]
        for h in range(_NH):
            o2 = jnp.swapaxes(heads[h], 1, 2).reshape(_NT, _D)
            proj = proj + jnp.dot(o2, towt[pl.ds(h * _D, _D), :],
                                  preferred_element_type=_F32)
        tm = _ln_lanes(proj + tx_s[...].reshape(_NT, _NINP), g1[...], b1[...])
        tm3 = tm.reshape(_T, 48, _NINP)
        for b in range(_B):
            for sp in range(_S):
                inp_s[pl.ds(b * _T, _T), pl.ds(sp * _NINP, _NINP)] = tm3[:, b * _S + sp, :]

    @pl.when(s == 2)
    def _():
        heads = _attend(qkvs_s[...], 64, _S, False)
        proj = sob[...]
        for h in range(_NH):
            o2 = jnp.swapaxes(heads[h], 1, 2).reshape(_NT, _D)
            proj = proj + jnp.dot(o2, sowt[pl.ds(h * _D, _D), :],
                                  preferred_element_type=_F32)
        sm = _ln_lanes(proj + sx_s[...].reshape(_NT, _NINP), g2[...], b2[...])
        sm3 = sm.reshape(_S, _TOK, _NINP)
        for sp in range(_S):
            inp_s[:, pl.ds(sp * _NINP, _NINP)] += sm3[sp]

    @pl.when(s == 3)
    def _():
        hacc_s[...] = jnp.zeros_like(hacc_s)

    @pl.when((s >= _P_FFB) & (s < _P_FFB + _FC))
    def _():
        h = _gelu(jnp.dot(inp_s[...], fb1w[...], preferred_element_type=_F32)
                  + fb1b[...])
        hacc_s[...] += jnp.dot(h, fb2w[...], preferred_element_type=_F32)

    @pl.when(s == _P_FFB + _FC - 1)
    def _():
        inp = inp_s[...]
        xr2 = inp + hacc_s[...] + fb2b[...]
        xr2_s[...] = xr2

        logits = jnp.dot(xr2, gw[...], preferred_element_type=_F32)
        mx = jnp.max(logits, axis=1, keepdims=True)
        ex = jnp.exp(logits - mx)
        se = jnp.sum(ex, axis=1, keepdims=True)
        probs = ex / se
        lse = mx + jnp.log(se)
        zl = jnp.mean(lse * lse) * np.float32(_Z)

        v1 = jnp.max(probs, axis=1, keepdims=True)
        m1r = (probs == v1).astype(_F32)
        c1 = jnp.dot(m1r, lt16[...], preferred_element_type=_F32)
        m1 = m1r * (c1 == 1.0).astype(_F32)
        probs2 = probs * (1.0 - m1)
        v2 = jnp.max(probs2, axis=1, keepdims=True)
        m2r = (probs2 == v2).astype(_F32)
        c2 = jnp.dot(m2r, lt16[...], preferred_element_type=_F32)
        m2 = m2r * (c2 == 1.0).astype(_F32) * (v2 > np.float32(_THRESH)).astype(_F32)

        density = jnp.mean(probs, axis=0, keepdims=True)
        d1m = jnp.mean(m1, axis=0, keepdims=True)
        bal = jnp.mean(density * d1m) * np.float32(_NE * _NE * _BAL)
        aux_o[...] = jnp.broadcast_to(bal + zl, (1, 1))

        pos1 = jnp.dot(l64[...], m1, preferred_element_type=_F32) - 1.0
        m1k = m1 * (pos1 < np.float32(_CAP)).astype(_F32)
        cnt1 = jnp.sum(m1, axis=0, keepdims=True)
        pos2 = jnp.dot(l64[...], m2, preferred_element_type=_F32) - 1.0 + cnt1
        m2k = m2 * (pos2 < np.float32(_CAP)).astype(_F32)

        e16v = e16[...]
        ci = (jax.lax.broadcasted_iota(jnp.int32, (_TOK, _NSLOT), 1) % _CAP
              ).astype(_F32)
        oh1 = (jnp.dot(pos1, e16v, preferred_element_type=_F32) == ci).astype(_F32)
        oh2 = (jnp.dot(pos2, e16v, preferred_element_type=_F32) == ci).astype(_F32)
        d1e = jnp.dot(m1k, e16v, preferred_element_type=_F32) * oh1
        d2e = jnp.dot(m2k, e16v, preferred_element_type=_F32) * oh2
        comb_s[...] = v1 * d1e + v2 * d2e
        disp = d1e + d2e
        ein_s[...] = jax.lax.dot_general(disp, xr2, (((0,), (0,)), ((), ())),
                                         preferred_element_type=_F32)

    @pl.when((s >= _P_EXP) & (s < _P_EXP + _NE))
    def _():
        e = s - _P_EXP
        slot = jax.lax.rem(e, _NS1)
        slot2 = jax.lax.rem(e, _NS2)
        row = pl.multiple_of(e * _CAP, _CAP)
        ein_e = ein_s[pl.ds(row, _CAP), :]

        pltpu.make_async_copy(ew1.at[0], w1buf.at[slot], sem1.at[slot]).wait()
        part = (jnp.dot(ein_e, w1buf[slot], preferred_element_type=_F32)
                + eb1[pl.ds(e, 1), :])
        act = part[:, :_HID] * _gelu(part[:, _HID:])

        pltpu.make_async_copy(ew2.at[0], w2buf.at[slot2], sem2.at[slot2]).wait()
        eo = (jnp.dot(act, w2buf[slot2], preferred_element_type=_F32)
              + eb2[pl.ds(e, 1)])
        eo_s[pl.ds(row, _CAP), :] = eo

        @pl.when(e + _NS1 < _NE)
        def _():
            pltpu.make_async_copy(ew1.at[e + _NS1], w1buf.at[slot],
                                  sem1.at[slot]).start()

        @pl.when(e + _NS2 < _NE)
        def _():
            pltpu.make_async_copy(ew2.at[e + _NS2], w2buf.at[slot2],
                                  sem2.at[slot2]).start()

    @pl.when(s == _P_FFA)
    def _():
        xr3_s[...] = xr2_s[...] + jnp.dot(comb_s[...], eo_s[...],
                                          preferred_element_type=_F32)
        hacc_s[...] = jnp.zeros_like(hacc_s)

    @pl.when(s >= _P_FFA)
    def _():
        h = _gelu(jnp.dot(xr3_s[...], fa1w[...], preferred_element_type=_F32)
                  + fa1b[...])
        hacc_s[...] += jnp.dot(h, fa2w[...], preferred_element_type=_F32)

    @pl.when(s == _NSTEP - 1)
    def _():
        z = xr3_s[...] + hacc_s[...] + fa2b[...] + inp_s[...]
        inv = np.float32(1.0 / _NINP)
        mu = jnp.dot(jnp.dot(z, g24[...], preferred_element_type=_F32) * inv,
                     g24t[...], preferred_element_type=_F32)
        d = z - mu
        var = jnp.dot(jnp.dot(d * d, g24[...], preferred_element_type=_F32) * inv,
                      g24t[...], preferred_element_type=_F32)
        y_o[...] = d * jax.lax.rsqrt(var + np.float32(1e-5)) * g3[...] + b3[...]


def kernel(x, t_in_w, t_in_b, t_out_w, t_out_b, s_in_w, s_in_b, s_out_w, s_out_b,
           ln1_g, ln1_b, ln2_g, ln2_b, ln3_g, ln3_b,
           ffb_w1, ffb_b1, ffb_w2, ffb_b2,
           gate_w, ew1, eb1, ew2, eb2,
           ffa_w1, ffa_b1, ffa_w2, ffa_b2):
    f32 = _F32

    l64 = jnp.tril(jnp.ones((_TOK, _TOK), f32))
    lt16 = jnp.triu(jnp.ones((_NE, _NE), f32))
    e16 = (jnp.arange(_NSLOT, dtype=jnp.int32)[None, :] // _CAP ==
           jnp.arange(_NE, dtype=jnp.int32)[:, None]).astype(f32)
    g24 = (jnp.arange(_DIM, dtype=jnp.int32)[:, None] // _NINP ==
           jnp.arange(_S, dtype=jnp.int32)[None, :]).astype(f32)
    g3 = jnp.tile(ln3_g, _S).reshape(1, _DIM)
    b3 = jnp.tile(ln3_b, _S).reshape(1, _DIM)

    cst = lambda *idx: (lambda s, _i=idx: _i)
    ffb_i = lambda s: (0, jnp.clip(s - _P_FFB, 0, _FC - 1))
    ffb_i2 = lambda s: (jnp.clip(s - _P_FFB, 0, _FC - 1), 0)
    ffa_i = lambda s: (0, jnp.clip(s - _P_FFA, 0, _FC - 1))
    ffa_i2 = lambda s: (jnp.clip(s - _P_FFA, 0, _FC - 1), 0)

    y, aux = pl.pallas_call(
        _mega_body,
        grid=(_NSTEP,),
        compiler_params=pltpu.CompilerParams(
            vmem_limit_bytes=100 * 1024 * 1024),
        in_specs=[
            pl.BlockSpec((_B, _T, _S, _NINP), cst(0, 0, 0, 0)),
            pl.BlockSpec((_NINP, 3 * _NINP), cst(0, 0)),
            pl.BlockSpec((1, 3 * _NINP), cst(0, 0)),
            pl.BlockSpec((_NINP, _NINP), cst(0, 0)),
            pl.BlockSpec((1, _NINP), cst(0, 0)),
            pl.BlockSpec((1, _NINP), cst(0, 0)),
            pl.BlockSpec((1, _NINP), cst(0, 0)),
            pl.BlockSpec((_NINP, 3 * _NINP), cst(0, 0)),
            pl.BlockSpec((1, 3 * _NINP), cst(0, 0)),
            pl.BlockSpec((_NINP, _NINP), cst(0, 0)),
            pl.BlockSpec((1, _NINP), cst(0, 0)),
            pl.BlockSpec((1, _NINP), cst(0, 0)),
            pl.BlockSpec((1, _NINP), cst(0, 0)),
            pl.BlockSpec((_DIM, _FW), ffb_i),
            pl.BlockSpec((1, _FW), ffb_i),
            pl.BlockSpec((_FW, _DIM), ffb_i2),
            pl.BlockSpec((1, _DIM), cst(0, 0)),
            pl.BlockSpec((_DIM, _NE), cst(0, 0)),
            pl.BlockSpec((_TOK, _TOK), cst(0, 0)),
            pl.BlockSpec((_NE, _NE), cst(0, 0)),
            pl.BlockSpec((_NE, _NSLOT), cst(0, 0)),
            pl.BlockSpec(memory_space=pl.ANY),
            pl.BlockSpec((_NE, 2 * _HID), cst(0, 0)),
            pl.BlockSpec(memory_space=pl.ANY),
            pl.BlockSpec((_NE, _DIM), cst(0, 0)),
            pl.BlockSpec((_DIM, _FW), ffa_i),
            pl.BlockSpec((1, _FW), ffa_i),
            pl.BlockSpec((_FW, _DIM), ffa_i2),
            pl.BlockSpec((1, _DIM), cst(0, 0)),
            pl.BlockSpec((_DIM, _S), cst(0, 0)),
            pl.BlockSpec((_S, _DIM), cst(0, 0)),
            pl.BlockSpec((1, _DIM), cst(0, 0)),
            pl.BlockSpec((1, _DIM), cst(0, 0)),
        ],
        out_specs=[pl.BlockSpec((_TOK, _DIM), cst(0, 0)),
                   pl.BlockSpec((1, 1), cst(0, 0))],
        out_shape=[jax.ShapeDtypeStruct((_TOK, _DIM), f32),
                   jax.ShapeDtypeStruct((1, 1), f32)],
        scratch_shapes=[
            pltpu.VMEM((_T, 48, _NINP), f32),
            pltpu.VMEM((_S, _TOK, _NINP), f32),
            pltpu.VMEM((_NT, 3 * _NINP), f32),
            pltpu.VMEM((_NT, 3 * _NINP), f32),
            pltpu.VMEM((_TOK, _DIM), f32),
            pltpu.VMEM((_TOK, _DIM), f32),
            pltpu.VMEM((_TOK, _DIM), f32),
            pltpu.VMEM((_NSLOT, _DIM), f32),
            pltpu.VMEM((_TOK, _NSLOT), f32),
            pltpu.VMEM((_NSLOT, _DIM), f32),
            pltpu.VMEM((_TOK, _DIM), f32),
            pltpu.VMEM((_CAP, _HID), f32),
            pltpu.VMEM((_NS1, _DIM, 2 * _HID), f32),
            pltpu.VMEM((_NS2, _HID, _DIM), f32),
            pltpu.SemaphoreType.DMA((_NS1,)),
            pltpu.SemaphoreType.DMA((_NS2,)),
        ],
    )(x, t_in_w.T, t_in_b.reshape(1, -1), t_out_w.T, t_out_b.reshape(1, -1),
      ln1_g.reshape(1, -1), ln1_b.reshape(1, -1),
      s_in_w.T, s_in_b.reshape(1, -1), s_out_w.T, s_out_b.reshape(1, -1),
      ln2_g.reshape(1, -1), ln2_b.reshape(1, -1),
      ffb_w1, ffb_b1.reshape(1, -1), ffb_w2, ffb_b2.reshape(1, -1),
      gate_w, l64, lt16, e16,
      ew1, eb1, ew2, eb2,
      ffa_w1, ffa_b1.reshape(1, -1), ffa_w2, ffa_b2.reshape(1, -1),
      g24, g24.T, g3, b3)

    return y.reshape(_B, _T, _S, _NINP), aux[0, 0]
